# baseline (speedup 1.0000x reference)
import functools

import jax
import jax.numpy as jnp
from jax import lax
from jax.experimental import pallas as pl
from jax.experimental.pallas import tpu as pltpu
from jax.experimental.pallas import tpu_sc as plsc

N_NODES = 10000
N_EDGES = 320000
D = 128

NC = 2
NS = 16
CHUNK = 128

NT = NC * NS
IBLK = 8
PAD_UNIT = NT * CHUNK * IBLK * 2
EDGES_PAD = ((N_EDGES + PAD_UNIT - 1) // PAD_UNIT) * PAD_UNIT
EDGES_PER_TILE = EDGES_PAD // NT
CPT = EDGES_PER_TILE // CHUNK
BLOCKS = CPT // IBLK

ACC_ROWS = 10240
ZERO_PER_TILE = ACC_ROWS // NS
WRITE_PER_TILE = (N_NODES // NS) // 8 * 8
WRITE_TAIL = N_NODES - NS * WRITE_PER_TILE


def _linear_body(feat_ref, w_ref, b_ref, out_ref):
    h = lax.dot_general(
        feat_ref[...], w_ref[...],
        dimension_numbers=(((1,), (1,)), ((), ())),
        preferred_element_type=jnp.float32,
    )
    out_ref[...] = h + b_ref[...]


def _linear(feat, W, b):
    rb = 1000
    return pl.pallas_call(
        _linear_body,
        grid=(N_NODES // rb,),
        in_specs=[
            pl.BlockSpec((rb, D), lambda i: (i, 0)),
            pl.BlockSpec((D, D), lambda i: (0, 0)),
            pl.BlockSpec((1, D), lambda i: (0, 0)),
        ],
        out_specs=pl.BlockSpec((rb, D), lambda i: (i, 0)),
        out_shape=jax.ShapeDtypeStruct((N_NODES, D), jnp.float32),
    )(feat, W, b.reshape(1, D))


def _mp_body(h, zeros, sd3, out, srcA, dstA, srcB, dstB,
             rows, acc, semG, semS, semSA, semDA, semSB, semDB):
    c = lax.axis_index("c")
    s = lax.axis_index("s")
    tid = c * NS + s

    z0 = s * ZERO_PER_TILE
    pltpu.sync_copy(zeros.at[pl.ds(z0, ZERO_PER_TILE)],
                    acc.at[pl.ds(z0, ZERO_PER_TILE)])

    def fire_g(idx_row, p):
        pltpu.async_copy(h.at[idx_row], rows[p], semG[p])

    def wait_g(p):
        pltpu.make_async_copy(h.at[srcA.at[0]], rows[p], semG[p]).wait()

    plsc.subcore_barrier()

    @pl.loop(0, CPT)
    def _(g):
        pltpu.sync_copy(sd3.at[tid, g], srcA)
        fire_g(srcA.at[0], 0)
        wait_g(0)
        pltpu.sync_copy(rows[0], acc.at[srcA.at[1]], add=True)

    plsc.subcore_barrier()
    w0 = s * WRITE_PER_TILE
    pltpu.sync_copy(acc.at[pl.ds(w0, WRITE_PER_TILE)],
                    out.at[c, pl.ds(w0, WRITE_PER_TILE)])

    @pl.when(s == 0)
    def _():
        t0 = NS * WRITE_PER_TILE
        pltpu.sync_copy(acc.at[pl.ds(t0, WRITE_TAIL)],
                        out.at[c, pl.ds(t0, WRITE_TAIL)])


@functools.partial(
    pl.kernel,
    out_type=jax.ShapeDtypeStruct((NC, N_NODES, D), jnp.float32),
    mesh=plsc.VectorSubcoreMesh(core_axis_name="c", subcore_axis_name="s"),
    scratch_types=[
        pltpu.VMEM((2, CHUNK), jnp.int32),
        pltpu.VMEM((2, CHUNK), jnp.int32),
        pltpu.VMEM((2, CHUNK), jnp.int32),
        pltpu.VMEM((2, CHUNK), jnp.int32),
        [pltpu.VMEM((CHUNK, D), jnp.float32)] * 2,
        pltpu.VMEM_SHARED((ACC_ROWS, D), jnp.float32),
        [pltpu.SemaphoreType.DMA] * 2,
        [pltpu.SemaphoreType.DMA] * 2,
        pltpu.SemaphoreType.DMA,
        pltpu.SemaphoreType.DMA,
        pltpu.SemaphoreType.DMA,
        pltpu.SemaphoreType.DMA,
    ],
)
def _message_passing(h, zeros, sd3, out, srcA, dstA, srcB, dstB,
                     rows, acc, semG, semS, semSA, semDA, semSB, semDB):
    _mp_body(h, zeros, sd3, out, srcA, dstA, srcB, dstB,
             rows, acc, semG, semS, semSA, semDA, semSB, semDB)


def _combine_body(p_ref, out_ref):
    out_ref[...] = p_ref[0] + p_ref[1]


def _combine(p):
    rb = 1000
    return pl.pallas_call(
        _combine_body,
        grid=(N_NODES // rb,),
        in_specs=[pl.BlockSpec((NC, rb, D), lambda i: (0, i, 0))],
        out_specs=pl.BlockSpec((rb, D), lambda i: (i, 0)),
        out_shape=jax.ShapeDtypeStruct((N_NODES, D), jnp.float32),
    )(p)


@jax.jit
def kernel(feat, edge_index, W, b):
    h = _linear(feat, W, b)
    npad = EDGES_PAD - N_EDGES
    src3 = jnp.concatenate(
        [edge_index[0], jnp.zeros((npad,), jnp.int32)]).reshape(NT, CPT, 1, CHUNK)
    dst3 = jnp.concatenate(
        [edge_index[1], jnp.full((npad,), N_NODES, jnp.int32)]
    ).reshape(NT, CPT, 1, CHUNK)
    sd3 = jnp.concatenate([src3, dst3], axis=2)
    zeros = jnp.zeros((ACC_ROWS, D), jnp.float32)
    return _combine(_message_passing(h, zeros, sd3))


# speedup vs baseline: 1.4816x; 1.4816x over previous
import functools

import jax
import jax.numpy as jnp
from jax import lax
from jax.experimental import pallas as pl
from jax.experimental.pallas import tpu as pltpu
from jax.experimental.pallas import tpu_sc as plsc

N_NODES = 10000
N_EDGES = 320000
D = 128

NC = 2
NS = 16
CHUNK = 128

NT = NC * NS
EDGES_PAD = ((N_EDGES + NT * CHUNK - 1) // (NT * CHUNK)) * (NT * CHUNK)
EDGES_PER_TILE = EDGES_PAD // NT
CHUNKS_PER_TILE = EDGES_PER_TILE // CHUNK

ACC_ROWS = 10240
ZERO_PER_TILE = ACC_ROWS // NS
WRITE_PER_TILE = (N_NODES // NS) // 8 * 8
WRITE_TAIL = N_NODES - NS * WRITE_PER_TILE


def _linear_body(feat_ref, w_ref, b_ref, out_ref):
    h = lax.dot_general(
        feat_ref[...], w_ref[...],
        dimension_numbers=(((1,), (1,)), ((), ())),
        preferred_element_type=jnp.float32,
    )
    out_ref[...] = h + b_ref[...]


def _linear(feat, W, b):
    rb = 1000
    return pl.pallas_call(
        _linear_body,
        grid=(N_NODES // rb,),
        in_specs=[
            pl.BlockSpec((rb, D), lambda i: (i, 0)),
            pl.BlockSpec((D, D), lambda i: (0, 0)),
            pl.BlockSpec((1, D), lambda i: (0, 0)),
        ],
        out_specs=pl.BlockSpec((rb, D), lambda i: (i, 0)),
        out_shape=jax.ShapeDtypeStruct((N_NODES, D), jnp.float32),
    )(feat, W, b.reshape(1, D))


def _mp_body(h, zeros, src, dst, out, src_v, dst_v, rows_v, acc, sem):
    c = lax.axis_index("c")
    s = lax.axis_index("s")

    z0 = s * ZERO_PER_TILE
    pltpu.sync_copy(zeros.at[pl.ds(z0, ZERO_PER_TILE)],
                    acc.at[pl.ds(z0, ZERO_PER_TILE)])
    plsc.subcore_barrier()

    base0 = (c * NS + s) * EDGES_PER_TILE

    @pl.loop(0, CHUNKS_PER_TILE)
    def _(g):
        base = base0 + g * CHUNK
        pltpu.sync_copy(src.at[pl.ds(base, CHUNK)], src_v)
        pltpu.sync_copy(dst.at[pl.ds(base, CHUNK)], dst_v)
        pltpu.async_copy(h.at[src_v], rows_v, sem).wait()
        pltpu.sync_copy(rows_v, acc.at[dst_v], add=True)

    plsc.subcore_barrier()
    w0 = s * WRITE_PER_TILE
    pltpu.sync_copy(acc.at[pl.ds(w0, WRITE_PER_TILE)],
                    out.at[c, pl.ds(w0, WRITE_PER_TILE)])

    @pl.when(s == 0)
    def _():
        t0 = NS * WRITE_PER_TILE
        pltpu.sync_copy(acc.at[pl.ds(t0, WRITE_TAIL)],
                        out.at[c, pl.ds(t0, WRITE_TAIL)])


@functools.partial(
    pl.kernel,
    out_type=jax.ShapeDtypeStruct((NC, N_NODES, D), jnp.float32),
    mesh=plsc.VectorSubcoreMesh(core_axis_name="c", subcore_axis_name="s"),
    scratch_types=[
        pltpu.VMEM((CHUNK,), jnp.int32),
        pltpu.VMEM((CHUNK,), jnp.int32),
        pltpu.VMEM((CHUNK, D), jnp.float32),
        pltpu.VMEM_SHARED((ACC_ROWS, D), jnp.float32),
        pltpu.SemaphoreType.DMA,
    ],
)
def _message_passing(h, zeros, src, dst, out, src_v, dst_v, rows_v, acc, sem):
    _mp_body(h, zeros, src, dst, out, src_v, dst_v, rows_v, acc, sem)


def _combine_body(p_ref, out_ref):
    out_ref[...] = p_ref[0] + p_ref[1]


def _combine(p):
    rb = 1000
    return pl.pallas_call(
        _combine_body,
        grid=(N_NODES // rb,),
        in_specs=[pl.BlockSpec((NC, rb, D), lambda i: (0, i, 0))],
        out_specs=pl.BlockSpec((rb, D), lambda i: (i, 0)),
        out_shape=jax.ShapeDtypeStruct((N_NODES, D), jnp.float32),
    )(p)


@jax.jit
def kernel(feat, edge_index, W, b):
    h = _linear(feat, W, b)
    npad = EDGES_PAD - N_EDGES
    src = jnp.concatenate([edge_index[0], jnp.zeros((npad,), jnp.int32)])
    dst = jnp.concatenate([edge_index[1], jnp.full((npad,), N_NODES, jnp.int32)])
    zeros = jnp.zeros((ACC_ROWS, D), jnp.float32)
    return _combine(_message_passing(h, zeros, src, dst))


# speedup vs baseline: 1.6026x; 1.0816x over previous
import functools

import jax
import jax.numpy as jnp
from jax import lax
from jax.experimental import pallas as pl
from jax.experimental.pallas import tpu as pltpu
from jax.experimental.pallas import tpu_sc as plsc

N_NODES = 10000
N_EDGES = 320000
D = 128

NC = 2
NS = 16
CHUNK = 128

NT = NC * NS
EDGES_PAD = ((N_EDGES + NT * CHUNK - 1) // (NT * CHUNK)) * (NT * CHUNK)
EDGES_PER_TILE = EDGES_PAD // NT
CHUNKS_PER_TILE = EDGES_PER_TILE // CHUNK

ACC_ROWS = 10240
ZERO_PER_TILE = ACC_ROWS // NS
WRITE_PER_TILE = (N_NODES // NS) // 8 * 8
WRITE_TAIL = N_NODES - NS * WRITE_PER_TILE


def _linear_body(feat_ref, w_ref, b_ref, out_ref):
    h = lax.dot_general(
        feat_ref[...], w_ref[...],
        dimension_numbers=(((1,), (1,)), ((), ())),
        preferred_element_type=jnp.float32,
    )
    out_ref[...] = h + b_ref[...]


def _linear(feat, W, b):
    rb = 1000
    return pl.pallas_call(
        _linear_body,
        grid=(N_NODES // rb,),
        in_specs=[
            pl.BlockSpec((rb, D), lambda i: (i, 0)),
            pl.BlockSpec((D, D), lambda i: (0, 0)),
            pl.BlockSpec((1, D), lambda i: (0, 0)),
        ],
        out_specs=pl.BlockSpec((rb, D), lambda i: (i, 0)),
        out_shape=jax.ShapeDtypeStruct((N_NODES, D), jnp.float32),
    )(feat, W, b.reshape(1, D))


def _mp_body(h, zeros, src, dst, out, src_v, dst_v, rows_v, acc, sem):
    c = lax.axis_index("c")
    s = lax.axis_index("s")

    z0 = s * ZERO_PER_TILE
    pltpu.sync_copy(zeros.at[pl.ds(z0, ZERO_PER_TILE)],
                    acc.at[pl.ds(z0, ZERO_PER_TILE)])
    plsc.subcore_barrier()

    base0 = (c * NS + s) * EDGES_PER_TILE

    @pl.loop(0, CHUNKS_PER_TILE)
    def _(g):
        base = base0 + g * CHUNK
        pltpu.sync_copy(src.at[pl.ds(base, CHUNK)], src_v)
        cp = pltpu.async_copy(h.at[src_v], rows_v, sem)
        pltpu.sync_copy(dst.at[pl.ds(base, CHUNK)], dst_v)
        cp.wait()
        pltpu.sync_copy(rows_v, acc.at[dst_v], add=True)

    plsc.subcore_barrier()
    w0 = s * WRITE_PER_TILE
    pltpu.sync_copy(acc.at[pl.ds(w0, WRITE_PER_TILE)],
                    out.at[c, pl.ds(w0, WRITE_PER_TILE)])

    @pl.when(s == 0)
    def _():
        t0 = NS * WRITE_PER_TILE
        pltpu.sync_copy(acc.at[pl.ds(t0, WRITE_TAIL)],
                        out.at[c, pl.ds(t0, WRITE_TAIL)])


@functools.partial(
    pl.kernel,
    out_type=jax.ShapeDtypeStruct((NC, N_NODES, D), jnp.float32),
    mesh=plsc.VectorSubcoreMesh(core_axis_name="c", subcore_axis_name="s"),
    scratch_types=[
        pltpu.VMEM((CHUNK,), jnp.int32),
        pltpu.VMEM((CHUNK,), jnp.int32),
        pltpu.VMEM((CHUNK, D), jnp.float32),
        pltpu.VMEM_SHARED((ACC_ROWS, D), jnp.float32),
        pltpu.SemaphoreType.DMA,
    ],
)
def _message_passing(h, zeros, src, dst, out, src_v, dst_v, rows_v, acc, sem):
    _mp_body(h, zeros, src, dst, out, src_v, dst_v, rows_v, acc, sem)


def _combine_body(p_ref, out_ref):
    out_ref[...] = p_ref[0] + p_ref[1]


def _combine(p):
    rb = 1000
    return pl.pallas_call(
        _combine_body,
        grid=(N_NODES // rb,),
        in_specs=[pl.BlockSpec((NC, rb, D), lambda i: (0, i, 0))],
        out_specs=pl.BlockSpec((rb, D), lambda i: (i, 0)),
        out_shape=jax.ShapeDtypeStruct((N_NODES, D), jnp.float32),
    )(p)


@jax.jit
def kernel(feat, edge_index, W, b):
    h = _linear(feat, W, b)
    npad = EDGES_PAD - N_EDGES
    src = jnp.concatenate([edge_index[0], jnp.zeros((npad,), jnp.int32)])
    dst = jnp.concatenate([edge_index[1], jnp.full((npad,), N_NODES, jnp.int32)])
    zeros = jnp.zeros((ACC_ROWS, D), jnp.float32)
    return _combine(_message_passing(h, zeros, src, dst))


# speedup vs baseline: 1.7263x; 1.0772x over previous
import functools

import jax
import jax.numpy as jnp
from jax import lax
from jax.experimental import pallas as pl
from jax.experimental.pallas import tpu as pltpu
from jax.experimental.pallas import tpu_sc as plsc

N_NODES = 10000
N_EDGES = 320000
D = 128

NC = 2
NS = 16
CHUNK = 128

NT = NC * NS
EDGES_PAD = ((N_EDGES + NT * CHUNK - 1) // (NT * CHUNK)) * (NT * CHUNK)
EDGES_PER_TILE = EDGES_PAD // NT
CHUNKS_PER_TILE = EDGES_PER_TILE // CHUNK

ACC_ROWS = 10240
ZERO_PER_TILE = ACC_ROWS // NS
WRITE_PER_TILE = (N_NODES // NS) // 8 * 8
WRITE_TAIL = N_NODES - NS * WRITE_PER_TILE


def _linear_body(feat_ref, w_ref, b_ref, out_ref):
    h = lax.dot_general(
        feat_ref[...], w_ref[...],
        dimension_numbers=(((1,), (1,)), ((), ())),
        preferred_element_type=jnp.float32,
    )
    out_ref[...] = h + b_ref[...]


def _linear(feat, W, b):
    rb = 1000
    return pl.pallas_call(
        _linear_body,
        grid=(N_NODES // rb,),
        in_specs=[
            pl.BlockSpec((rb, D), lambda i: (i, 0)),
            pl.BlockSpec((D, D), lambda i: (0, 0)),
            pl.BlockSpec((1, D), lambda i: (0, 0)),
        ],
        out_specs=pl.BlockSpec((rb, D), lambda i: (i, 0)),
        out_shape=jax.ShapeDtypeStruct((N_NODES, D), jnp.float32),
    )(feat, W, b.reshape(1, D))


def _mp_body(h, zeros, src, dst, out, src_v, dst_v, rows_v, acc, sem, semS):
    c = lax.axis_index("c")
    s = lax.axis_index("s")

    z0 = s * ZERO_PER_TILE
    pltpu.sync_copy(zeros.at[pl.ds(z0, ZERO_PER_TILE)],
                    acc.at[pl.ds(z0, ZERO_PER_TILE)])
    plsc.subcore_barrier()

    base0 = (c * NS + s) * EDGES_PER_TILE

    pltpu.sync_copy(dst.at[pl.ds(base0, CHUNK)], dst_v)
    pltpu.sync_copy(zeros.at[pl.ds(0, CHUNK)], rows_v)
    pltpu.async_copy(rows_v, acc.at[dst_v], semS, add=True)

    @pl.loop(0, CHUNKS_PER_TILE)
    def _(g):
        base = base0 + g * CHUNK
        pltpu.sync_copy(src.at[pl.ds(base, CHUNK)], src_v)
        pltpu.make_async_copy(rows_v, acc.at[dst_v], semS).wait()
        cp = pltpu.async_copy(h.at[src_v], rows_v, sem)
        pltpu.sync_copy(dst.at[pl.ds(base, CHUNK)], dst_v)
        cp.wait()
        pltpu.async_copy(rows_v, acc.at[dst_v], semS, add=True)

    pltpu.make_async_copy(rows_v, acc.at[dst_v], semS).wait()
    plsc.subcore_barrier()
    w0 = s * WRITE_PER_TILE
    pltpu.sync_copy(acc.at[pl.ds(w0, WRITE_PER_TILE)],
                    out.at[c, pl.ds(w0, WRITE_PER_TILE)])

    @pl.when(s == 0)
    def _():
        t0 = NS * WRITE_PER_TILE
        pltpu.sync_copy(acc.at[pl.ds(t0, WRITE_TAIL)],
                        out.at[c, pl.ds(t0, WRITE_TAIL)])


@functools.partial(
    pl.kernel,
    out_type=jax.ShapeDtypeStruct((NC, N_NODES, D), jnp.float32),
    mesh=plsc.VectorSubcoreMesh(core_axis_name="c", subcore_axis_name="s"),
    scratch_types=[
        pltpu.VMEM((CHUNK,), jnp.int32),
        pltpu.VMEM((CHUNK,), jnp.int32),
        pltpu.VMEM((CHUNK, D), jnp.float32),
        pltpu.VMEM_SHARED((ACC_ROWS, D), jnp.float32),
        pltpu.SemaphoreType.DMA,
        pltpu.SemaphoreType.DMA,
    ],
)
def _message_passing(h, zeros, src, dst, out, src_v, dst_v, rows_v, acc,
                     sem, semS):
    _mp_body(h, zeros, src, dst, out, src_v, dst_v, rows_v, acc, sem, semS)


def _combine_body(p_ref, out_ref):
    out_ref[...] = p_ref[0] + p_ref[1]


def _combine(p):
    rb = 1000
    return pl.pallas_call(
        _combine_body,
        grid=(N_NODES // rb,),
        in_specs=[pl.BlockSpec((NC, rb, D), lambda i: (0, i, 0))],
        out_specs=pl.BlockSpec((rb, D), lambda i: (i, 0)),
        out_shape=jax.ShapeDtypeStruct((N_NODES, D), jnp.float32),
    )(p)


@jax.jit
def kernel(feat, edge_index, W, b):
    h = _linear(feat, W, b)
    npad = EDGES_PAD - N_EDGES
    src = jnp.concatenate([edge_index[0], jnp.zeros((npad,), jnp.int32)])
    dst = jnp.concatenate([edge_index[1], jnp.full((npad,), N_NODES, jnp.int32)])
    zeros = jnp.zeros((ACC_ROWS, D), jnp.float32)
    return _combine(_message_passing(h, zeros, src, dst))
